# baseline (speedup 1.0000x reference)
import functools

import jax
import jax.numpy as jnp
from jax import lax
from jax.experimental import pallas as pl
from jax.experimental.pallas import tpu as pltpu
from jax.experimental.pallas import tpu_sc as plsc

BATCH = 16384
DIM = 32
NUM_CORES = 2
NUM_SUBCORES = 16
LANES = 16
NUM_WORKERS = NUM_CORES * NUM_SUBCORES
BPW = BATCH // NUM_WORKERS
CHUNK = 128
NCHUNK = BPW // CHUNK
GROUPS = BPW // LANES


def _sc_body(pos_ci, pos_ei, neg_ci, neg_ei, case_emb, ent_emb,
             pos_out, neg_out,
             idx_pc, idx_pe, idx_nc, idx_ne,
             rows_pc, rows_pe, rows_nc, rows_ne,
             score_p, score_n, prod,
             sem_pc, sem_pe, sem_nc, sem_ne):
  wid = lax.axis_index("s") * NUM_CORES + lax.axis_index("c")
  base = wid * BPW

  for k in range(NCHUNK):
    sl = pl.ds(base + k * CHUNK, CHUNK)
    pltpu.sync_copy(pos_ci.at[sl], idx_pc.at[k])
    pltpu.sync_copy(pos_ei.at[sl], idx_pe.at[k])
    pltpu.sync_copy(neg_ci.at[sl], idx_nc.at[k])
    pltpu.sync_copy(neg_ei.at[sl], idx_ne.at[k])

  def fire(table, idx, rows, sem):
    return [
        pltpu.make_async_copy(table.at[idx.at[k]],
                              rows.at[pl.ds(k * CHUNK, CHUNK)], sem)
        for k in range(NCHUNK)
    ]

  cps_pc = fire(case_emb, idx_pc, rows_pc, sem_pc)
  cps_pe = fire(ent_emb, idx_pe, rows_pe, sem_pe)
  cps_nc = fire(case_emb, idx_nc, rows_nc, sem_nc)
  cps_ne = fire(ent_emb, idx_ne, rows_ne, sem_ne)
  for cps in (cps_pc, cps_pe, cps_nc, cps_ne):
    for cp in cps:
      cp.start()

  lane_iota = lax.iota(jnp.int32, LANES)

  def dot_groups(rows_a, rows_b, score, prod):
    def body(g, carry):
      base_r = g * LANES
      for j in range(LANES):
        r = base_r + j
        a_lo = rows_a[r, pl.ds(0, LANES)]
        a_hi = rows_a[r, pl.ds(LANES, LANES)]
        b_lo = rows_b[r, pl.ds(0, LANES)]
        b_hi = rows_b[r, pl.ds(LANES, LANES)]
        p = a_lo * b_lo + a_hi * b_hi
        plsc.store_scatter(prod, [lane_iota * LANES + j], p)
      acc = prod[pl.ds(0, LANES)]
      for d in range(1, LANES):
        acc = acc + prod[pl.ds(d * LANES, LANES)]
      score[pl.ds(base_r, LANES)] = acc
      return carry
    lax.fori_loop(0, GROUPS, body, 0, unroll=False)

  for cp in cps_pc + cps_pe:
    cp.wait()
  dot_groups(rows_pc, rows_pe, score_p, prod)
  pltpu.sync_copy(score_p, pos_out.at[pl.ds(base, BPW)])

  for cp in cps_nc + cps_ne:
    cp.wait()
  dot_groups(rows_nc, rows_ne, score_n, prod)
  pltpu.sync_copy(score_n, neg_out.at[pl.ds(base, BPW)])


_sc_scores = functools.partial(
    pl.kernel,
    out_type=[
        jax.ShapeDtypeStruct((BATCH,), jnp.float32),
        jax.ShapeDtypeStruct((BATCH,), jnp.float32),
    ],
    mesh=plsc.VectorSubcoreMesh(
        core_axis_name="c", subcore_axis_name="s",
        num_cores=NUM_CORES, num_subcores=NUM_SUBCORES),
    compiler_params=pltpu.CompilerParams(
        needs_layout_passes=False, use_tc_tiling_on_sc=False),
    scratch_types=[
        pltpu.VMEM((NCHUNK, CHUNK), jnp.int32),
        pltpu.VMEM((NCHUNK, CHUNK), jnp.int32),
        pltpu.VMEM((NCHUNK, CHUNK), jnp.int32),
        pltpu.VMEM((NCHUNK, CHUNK), jnp.int32),
        pltpu.VMEM((BPW, DIM), jnp.float32),
        pltpu.VMEM((BPW, DIM), jnp.float32),
        pltpu.VMEM((BPW, DIM), jnp.float32),
        pltpu.VMEM((BPW, DIM), jnp.float32),
        pltpu.VMEM((BPW,), jnp.float32),
        pltpu.VMEM((BPW,), jnp.float32),
        pltpu.VMEM((LANES * LANES,), jnp.float32),
        pltpu.SemaphoreType.DMA,
        pltpu.SemaphoreType.DMA,
        pltpu.SemaphoreType.DMA,
        pltpu.SemaphoreType.DMA,
    ],
)(_sc_body)


def _tc_loss_body(pos_ref, neg_ref, out_ref):
  ps = pos_ref[...]
  ns = neg_ref[...]

  def logsig(x):
    return jnp.minimum(x, 0.0) - jnp.log1p(jnp.exp(-jnp.abs(x)))

  total = jnp.sum(logsig(ps)) + jnp.sum(logsig(-ns))
  out_ref[0, 0] = -total


def kernel(pos_caseid, pos_entityid, neg_caseid, neg_entity,
           case_emb, entity_emb):
  pos_scores, neg_scores = _sc_scores(
      pos_caseid.astype(jnp.int32),
      pos_entityid.astype(jnp.int32),
      neg_caseid.astype(jnp.int32),
      neg_entity.astype(jnp.int32),
      case_emb, entity_emb)

  loss = pl.pallas_call(
      _tc_loss_body,
      out_shape=jax.ShapeDtypeStruct((1, 1), jnp.float32),
      out_specs=pl.BlockSpec(memory_space=pltpu.SMEM),
  )(pos_scores.reshape(128, 128), neg_scores.reshape(128, 128))
  return loss[0, 0]
